# baseline (speedup 1.0000x reference)
import functools

import jax
import jax.numpy as jnp
from jax import lax
from jax.experimental import pallas as pl
from jax.experimental.pallas import tpu as pltpu
from jax.experimental.pallas import tpu_sc as plsc

_NUM_WORKERS = 32
_IDX_LANES = 128
_CHUNK = 512


def _sc_gather(ids_2d, table2, n_rows):
    rows_per_w = n_rows // _NUM_WORKERS
    k = _CHUNK // _IDX_LANES
    n_outer = rows_per_w // (2 * _CHUNK)
    mesh = plsc.VectorSubcoreMesh(core_axis_name="c", subcore_axis_name="s")

    @functools.partial(
        pl.kernel,
        mesh=mesh,
        out_type=jax.ShapeDtypeStruct((n_rows, 128), jnp.float32),
        scratch_types=[
            pltpu.VMEM((2 * k, _IDX_LANES), jnp.int32),
            pltpu.VMEM((_CHUNK, 128), jnp.float32),
            pltpu.SemaphoreType.DMA,
        ],
    )
    def gather_kernel(ids_hbm, table_hbm, out_hbm, idx_v, rows_v, sem):
        wid = lax.axis_index("s") * 2 + lax.axis_index("c")
        base = wid * rows_per_w
        for c in range(n_outer):
            off = pl.multiple_of(base + c * 2 * _CHUNK, 2 * _CHUNK)
            pltpu.sync_copy(
                ids_hbm.at[pl.ds(pl.multiple_of(off // _IDX_LANES, 8), 2 * k)],
                idx_v)
            for half in range(2):
                copies = [
                    pltpu.async_copy(
                        table_hbm.at[idx_v.at[half * k + j]],
                        rows_v.at[pl.ds(j * _IDX_LANES, _IDX_LANES)],
                        sem,
                    )
                    for j in range(k)
                ]
                for cp in copies:
                    cp.wait()
                pltpu.sync_copy(
                    rows_v,
                    out_hbm.at[pl.ds(
                        pl.multiple_of(off + half * _CHUNK, _CHUNK), _CHUNK)])

    return gather_kernel(ids_2d, table2)


def _gelu(x):
    return 0.5 * x * (1.0 + lax.erf(x * 0.7071067811865476))


def _tc_body(ae_ref, aid_ref, rid_ref, sid_ref, st_ref, rt_ref, stt_ref,
             w1_ref, b1_ref, w2_ref, b2_ref, w3_ref, b3_ref, w4_ref, b4_ref,
             out_ref):
    f32 = jnp.float32
    dn = lambda c: ((c, ((), ())))
    ae = ae_ref[...]
    m1 = (aid_ref[0] & 1).astype(f32)
    m0 = 1.0 - m1
    rid = rid_ref[0]
    sid = sid_ref[0]
    st = st_ref[0]
    w3 = w3_ref[...]

    g = _gelu(w1_ref[...] * st + b1_ref[...])
    a1 = lax.dot_general(w2_ref[...], g, dn(((0,), (0,))),
                         preferred_element_type=f32) + b2_ref[...]

    rt3 = lax.dot_general(rt_ref[...], w3[64:80], dn(((1,), (0,))),
                          preferred_element_type=f32)
    st3 = lax.dot_general(stt_ref[...], w3[80:96], dn(((1,), (0,))),
                          preferred_element_type=f32)
    oh_r = (lax.broadcasted_iota(jnp.int32, (6, 1), 0) == rid).astype(f32)
    oh_s = (lax.broadcasted_iota(jnp.int32, (5, 1), 0) == sid).astype(f32)

    w3a = w3[0:64]
    c0 = lax.dot_general(w3a, ae[:, 0:64], dn(((0,), (1,))),
                         preferred_element_type=f32)
    c1 = lax.dot_general(w3a, ae[:, 64:128], dn(((0,), (1,))),
                         preferred_element_type=f32)
    pre = m0 * c0 + m1 * c1
    pre = pre + lax.dot_general(rt3, oh_r, dn(((0,), (0,))),
                                preferred_element_type=f32)
    pre = pre + lax.dot_general(st3, oh_s, dn(((0,), (0,))),
                                preferred_element_type=f32)
    pre = pre + lax.dot_general(w3[96:112], a1, dn(((0,), (0,))),
                                preferred_element_type=f32)
    pre = pre + b3_ref[...]
    h = _gelu(pre)
    out_ref[...] = lax.dot_general(h, w4_ref[...], dn(((0,), (0,))),
                                   preferred_element_type=f32) + b4_ref[...]


_R = 2048


def _tc_specs(nblk, h_dim):
    in_specs = [
        pl.BlockSpec((_R, 128), lambda i: (i, 0)),
        pl.BlockSpec((1, 1, _R), lambda i: (i, 0, 0)),
        pl.BlockSpec((1, 1, _R), lambda i: (i, 0, 0)),
        pl.BlockSpec((1, 1, _R), lambda i: (i, 0, 0)),
        pl.BlockSpec((1, 1, _R), lambda i: (i, 0, 0)),
        pl.BlockSpec((6, 16), lambda i: (0, 0)),
        pl.BlockSpec((5, 16), lambda i: (0, 0)),
        pl.BlockSpec((16, 1), lambda i: (0, 0)),
        pl.BlockSpec((16, 1), lambda i: (0, 0)),
        pl.BlockSpec((16, 16), lambda i: (0, 0)),
        pl.BlockSpec((16, 1), lambda i: (0, 0)),
        pl.BlockSpec((112, h_dim), lambda i: (0, 0)),
        pl.BlockSpec((h_dim, 1), lambda i: (0, 0)),
        pl.BlockSpec((h_dim, h_dim), lambda i: (0, 0)),
        pl.BlockSpec((1, h_dim), lambda i: (0, 0)),
    ]
    out_specs = pl.BlockSpec((_R, h_dim), lambda i: (i, 0))
    return in_specs, out_specs


def kernel(actor_ids, role_types, status_ids, screen_time, actor_table,
           role_table, status_table, W1, b1, W2, b2, W3, b3, W4, b4):
    b_sz, l_sz = actor_ids.shape
    n = b_sz * l_sz
    d = actor_table.shape[1]
    h_dim = W4.shape[1]
    nblk = n // _R

    ids_t = actor_ids.T.astype(jnp.int32)
    table2 = jnp.concatenate([actor_table[0::2], actor_table[1::2]], axis=1)
    ids_2d = lax.shift_right_logical(ids_t, 1).reshape(n // _IDX_LANES,
                                                       _IDX_LANES)
    actor_emb = _sc_gather(ids_2d, table2, n)

    aid3 = ids_t.reshape(nblk, 1, _R)
    rid3 = role_types.T.astype(jnp.int32).reshape(nblk, 1, _R)
    sid3 = status_ids.T.astype(jnp.int32).reshape(nblk, 1, _R)
    st3 = screen_time.T.astype(jnp.float32).reshape(nblk, 1, _R)

    in_specs, out_specs = _tc_specs(nblk, h_dim)
    out = pl.pallas_call(
        _tc_body,
        grid=(nblk,),
        in_specs=in_specs,
        out_specs=out_specs,
        out_shape=jax.ShapeDtypeStruct((n, h_dim), jnp.float32),
    )(actor_emb, aid3, rid3, sid3, st3, role_table, status_table,
      W1.reshape(d // 4, 1), b1.reshape(d // 4, 1), W2,
      b2.reshape(d // 4, 1), W3, b3.reshape(h_dim, 1), W4,
      b4.reshape(1, h_dim))
    return out.reshape(l_sz, b_sz, h_dim).transpose(1, 0, 2)


# speedup vs baseline: 7.2577x; 7.2577x over previous
import functools

import jax
import jax.numpy as jnp
from jax import lax
from jax.experimental import pallas as pl
from jax.experimental.pallas import tpu as pltpu
from jax.experimental.pallas import tpu_sc as plsc

_NUM_WORKERS = 32
_IDX_LANES = 128
_CHUNK = 512


def _sc_gather(ids_2d, table2, n_rows):
    rows_per_w = n_rows // _NUM_WORKERS
    k = _CHUNK // _IDX_LANES
    n_outer = rows_per_w // (2 * _CHUNK)
    mesh = plsc.VectorSubcoreMesh(core_axis_name="c", subcore_axis_name="s")

    @functools.partial(
        pl.kernel,
        mesh=mesh,
        out_type=jax.ShapeDtypeStruct((n_rows, 128), jnp.float32),
        scratch_types=[
            pltpu.VMEM((2 * k, _IDX_LANES), jnp.int32),
            pltpu.VMEM((_CHUNK, 128), jnp.float32),
            pltpu.SemaphoreType.DMA,
        ],
    )
    def gather_kernel(ids_hbm, table_hbm, out_hbm, idx_v, rows_v, sem):
        wid = lax.axis_index("s") * 2 + lax.axis_index("c")
        base = wid * rows_per_w
        for c in range(n_outer):
            off = pl.multiple_of(base + c * 2 * _CHUNK, 2 * _CHUNK)
            pltpu.sync_copy(
                ids_hbm.at[pl.ds(pl.multiple_of(off // _IDX_LANES, 8), 2 * k)],
                idx_v)
            for half in range(2):
                copies = [
                    pltpu.async_copy(
                        table_hbm.at[idx_v.at[half * k + j]],
                        rows_v.at[pl.ds(j * _IDX_LANES, _IDX_LANES)],
                        sem,
                    )
                    for j in range(k)
                ]
                for cp in copies:
                    cp.wait()
                pltpu.sync_copy(
                    rows_v,
                    out_hbm.at[pl.ds(
                        pl.multiple_of(off + half * _CHUNK, _CHUNK), _CHUNK)])

    return gather_kernel(ids_2d, table2)


_PAIR_C = 512
_SPLIT = 977 * _PAIR_C


def _pair_body(ta_ref, tb_ref, out_ref):
    out_ref[...] = jnp.concatenate(
        [jnp.transpose(ta_ref[...]), jnp.transpose(tb_ref[...])], axis=1)


def _pair_table(actor_table):
    v, d = actor_table.shape
    tt = actor_table.T
    grid = _SPLIT // _PAIR_C
    return pl.pallas_call(
        _pair_body,
        grid=(grid,),
        in_specs=[
            pl.BlockSpec((d, _PAIR_C), lambda i: (0, i)),
            pl.BlockSpec((d, _PAIR_C), lambda i: (0, i + _SPLIT // _PAIR_C)),
        ],
        out_specs=pl.BlockSpec((_PAIR_C, 2 * d), lambda i: (i, 0)),
        out_shape=jax.ShapeDtypeStruct((_SPLIT, 2 * d), jnp.float32),
    )(tt, tt)


def _gelu(x):
    return 0.5 * x * (1.0 + lax.erf(x * 0.7071067811865476))


def _tc_body(ae_ref, aid_ref, rid_ref, sid_ref, st_ref, rt_ref, stt_ref,
             w1_ref, b1_ref, w2_ref, b2_ref, w3_ref, b3_ref, w4_ref, b4_ref,
             out_ref):
    f32 = jnp.float32
    dn = lambda c: ((c, ((), ())))
    ae = ae_ref[...]
    m1 = (aid_ref[0] >= _SPLIT).astype(f32)
    m0 = 1.0 - m1
    rid = rid_ref[0]
    sid = sid_ref[0]
    st = st_ref[0]
    w3 = w3_ref[...]

    g = _gelu(w1_ref[...] * st + b1_ref[...])
    a1 = lax.dot_general(w2_ref[...], g, dn(((0,), (0,))),
                         preferred_element_type=f32) + b2_ref[...]

    rt3 = lax.dot_general(rt_ref[...], w3[64:80], dn(((1,), (0,))),
                          preferred_element_type=f32)
    st3 = lax.dot_general(stt_ref[...], w3[80:96], dn(((1,), (0,))),
                          preferred_element_type=f32)
    oh_r = (lax.broadcasted_iota(jnp.int32, (6, 1), 0) == rid).astype(f32)
    oh_s = (lax.broadcasted_iota(jnp.int32, (5, 1), 0) == sid).astype(f32)

    w3a = w3[0:64]
    c0 = lax.dot_general(w3a, ae[:, 0:64], dn(((0,), (1,))),
                         preferred_element_type=f32)
    c1 = lax.dot_general(w3a, ae[:, 64:128], dn(((0,), (1,))),
                         preferred_element_type=f32)
    pre = m0 * c0 + m1 * c1
    pre = pre + lax.dot_general(rt3, oh_r, dn(((0,), (0,))),
                                preferred_element_type=f32)
    pre = pre + lax.dot_general(st3, oh_s, dn(((0,), (0,))),
                                preferred_element_type=f32)
    pre = pre + lax.dot_general(w3[96:112], a1, dn(((0,), (0,))),
                                preferred_element_type=f32)
    pre = pre + b3_ref[...]
    h = _gelu(pre)
    out_ref[...] = lax.dot_general(h, w4_ref[...], dn(((0,), (0,))),
                                   preferred_element_type=f32) + b4_ref[...]


_R = 2048


def _tc_specs(nblk, h_dim):
    in_specs = [
        pl.BlockSpec((_R, 128), lambda i: (i, 0)),
        pl.BlockSpec((1, 1, _R), lambda i: (i, 0, 0)),
        pl.BlockSpec((1, 1, _R), lambda i: (i, 0, 0)),
        pl.BlockSpec((1, 1, _R), lambda i: (i, 0, 0)),
        pl.BlockSpec((1, 1, _R), lambda i: (i, 0, 0)),
        pl.BlockSpec((6, 16), lambda i: (0, 0)),
        pl.BlockSpec((5, 16), lambda i: (0, 0)),
        pl.BlockSpec((16, 1), lambda i: (0, 0)),
        pl.BlockSpec((16, 1), lambda i: (0, 0)),
        pl.BlockSpec((16, 16), lambda i: (0, 0)),
        pl.BlockSpec((16, 1), lambda i: (0, 0)),
        pl.BlockSpec((112, h_dim), lambda i: (0, 0)),
        pl.BlockSpec((h_dim, 1), lambda i: (0, 0)),
        pl.BlockSpec((h_dim, h_dim), lambda i: (0, 0)),
        pl.BlockSpec((1, h_dim), lambda i: (0, 0)),
    ]
    out_specs = pl.BlockSpec((_R, h_dim), lambda i: (i, 0))
    return in_specs, out_specs


def kernel(actor_ids, role_types, status_ids, screen_time, actor_table,
           role_table, status_table, W1, b1, W2, b2, W3, b3, W4, b4):
    b_sz, l_sz = actor_ids.shape
    n = b_sz * l_sz
    d = actor_table.shape[1]
    h_dim = W4.shape[1]
    nblk = n // _R

    ids_t = actor_ids.T.astype(jnp.int32)
    table2 = _pair_table(actor_table)
    ids_2d = (ids_t - _SPLIT * (ids_t >= _SPLIT)).reshape(n // _IDX_LANES,
                                                          _IDX_LANES)
    actor_emb = _sc_gather(ids_2d, table2, n)

    aid3 = ids_t.reshape(nblk, 1, _R)
    rid3 = role_types.T.astype(jnp.int32).reshape(nblk, 1, _R)
    sid3 = status_ids.T.astype(jnp.int32).reshape(nblk, 1, _R)
    st3 = screen_time.T.astype(jnp.float32).reshape(nblk, 1, _R)

    in_specs, out_specs = _tc_specs(nblk, h_dim)
    out = pl.pallas_call(
        _tc_body,
        grid=(nblk,),
        in_specs=in_specs,
        out_specs=out_specs,
        out_shape=jax.ShapeDtypeStruct((n, h_dim), jnp.float32),
    )(actor_emb, aid3, rid3, sid3, st3, role_table, status_table,
      W1.reshape(d // 4, 1), b1.reshape(d // 4, 1), W2,
      b2.reshape(d // 4, 1), W3, b3.reshape(h_dim, 1), W4,
      b4.reshape(1, h_dim))
    return out.reshape(l_sz, b_sz, h_dim).transpose(1, 0, 2)


# speedup vs baseline: 11.2748x; 1.5535x over previous
import functools

import jax
import jax.numpy as jnp
from jax import lax
from jax.experimental import pallas as pl
from jax.experimental.pallas import tpu as pltpu
from jax.experimental.pallas import tpu_sc as plsc

_NUM_WORKERS = 32
_IDX_LANES = 128
_CHUNK = 512


def _sc_gather(ids_2d, table2, n_rows):
    rows_per_w = n_rows // _NUM_WORKERS
    k = _CHUNK // _IDX_LANES
    n_outer = rows_per_w // (2 * _CHUNK)
    mesh = plsc.VectorSubcoreMesh(core_axis_name="c", subcore_axis_name="s")

    @functools.partial(
        pl.kernel,
        mesh=mesh,
        out_type=jax.ShapeDtypeStruct((n_rows, 128), jnp.float32),
        scratch_types=[
            pltpu.VMEM((2 * k, _IDX_LANES), jnp.int32),
            pltpu.VMEM((_CHUNK, 128), jnp.float32),
            pltpu.SemaphoreType.DMA,
        ],
    )
    def gather_kernel(ids_hbm, table_hbm, out_hbm, idx_v, rows_v, sem):
        wid = lax.axis_index("s") * 2 + lax.axis_index("c")
        base = wid * rows_per_w
        for c in range(n_outer):
            off = pl.multiple_of(base + c * 2 * _CHUNK, 2 * _CHUNK)
            pltpu.sync_copy(
                ids_hbm.at[pl.ds(pl.multiple_of(off // _IDX_LANES, 8), 2 * k)],
                idx_v)
            for half in range(2):
                copies = [
                    pltpu.async_copy(
                        table_hbm.at[idx_v.at[half * k + j]],
                        rows_v.at[pl.ds(j * _IDX_LANES, _IDX_LANES)],
                        sem,
                    )
                    for j in range(k)
                ]
                for cp in copies:
                    cp.wait()
                pltpu.sync_copy(
                    rows_v,
                    out_hbm.at[pl.ds(
                        pl.multiple_of(off + half * _CHUNK, _CHUNK), _CHUNK)])

    return gather_kernel(ids_2d, table2)


_PAIR_C = 8192
_SPLIT = 62 * _PAIR_C


def _pair_body(ta_ref, tb_ref, out_ref):
    out_ref[...] = jnp.concatenate(
        [jnp.transpose(ta_ref[...]), jnp.transpose(tb_ref[...])], axis=1)


def _pair_table(actor_table):
    v, d = actor_table.shape
    tt = actor_table.T
    grid = _SPLIT // _PAIR_C
    return pl.pallas_call(
        _pair_body,
        grid=(grid,),
        in_specs=[
            pl.BlockSpec((d, _PAIR_C), lambda i: (0, i)),
            pl.BlockSpec(
                (d, _PAIR_C),
                lambda i: (0, jnp.minimum(i + _SPLIT // _PAIR_C,
                                          v // _PAIR_C))),
        ],
        out_specs=pl.BlockSpec((_PAIR_C, 2 * d), lambda i: (i, 0)),
        out_shape=jax.ShapeDtypeStruct((_SPLIT, 2 * d), jnp.float32),
    )(tt, tt)


def _gelu(x):
    return 0.5 * x * (1.0 + lax.erf(x * 0.7071067811865476))


def _tc_body(ae_ref, aid_ref, rid_ref, sid_ref, st_ref, rt_ref, stt_ref,
             w1_ref, b1_ref, w2_ref, b2_ref, w3_ref, b3_ref, w4_ref, b4_ref,
             out_ref):
    f32 = jnp.float32
    dn = lambda c: ((c, ((), ())))
    ae = ae_ref[...]
    m1 = (aid_ref[0] >= _SPLIT).astype(f32)
    m0 = 1.0 - m1
    rid = rid_ref[0]
    sid = sid_ref[0]
    st = st_ref[0]
    w3 = w3_ref[...]

    g = _gelu(w1_ref[...] * st + b1_ref[...])
    a1 = lax.dot_general(w2_ref[...], g, dn(((0,), (0,))),
                         preferred_element_type=f32) + b2_ref[...]

    rt3 = lax.dot_general(rt_ref[...], w3[64:80], dn(((1,), (0,))),
                          preferred_element_type=f32)
    st3 = lax.dot_general(stt_ref[...], w3[80:96], dn(((1,), (0,))),
                          preferred_element_type=f32)
    oh_r = (lax.broadcasted_iota(jnp.int32, (6, 1), 0) == rid).astype(f32)
    oh_s = (lax.broadcasted_iota(jnp.int32, (5, 1), 0) == sid).astype(f32)

    w3a = w3[0:64]
    c0 = lax.dot_general(w3a, ae[:, 0:64], dn(((0,), (1,))),
                         preferred_element_type=f32)
    c1 = lax.dot_general(w3a, ae[:, 64:128], dn(((0,), (1,))),
                         preferred_element_type=f32)
    pre = m0 * c0 + m1 * c1
    pre = pre + lax.dot_general(rt3, oh_r, dn(((0,), (0,))),
                                preferred_element_type=f32)
    pre = pre + lax.dot_general(st3, oh_s, dn(((0,), (0,))),
                                preferred_element_type=f32)
    pre = pre + lax.dot_general(w3[96:112], a1, dn(((0,), (0,))),
                                preferred_element_type=f32)
    pre = pre + b3_ref[...]
    h = _gelu(pre)
    out_ref[...] = lax.dot_general(h, w4_ref[...], dn(((0,), (0,))),
                                   preferred_element_type=f32) + b4_ref[...]


_R = 2048


def _tc_specs(nblk, h_dim):
    in_specs = [
        pl.BlockSpec((_R, 128), lambda i: (i, 0)),
        pl.BlockSpec((1, 1, _R), lambda i: (i, 0, 0)),
        pl.BlockSpec((1, 1, _R), lambda i: (i, 0, 0)),
        pl.BlockSpec((1, 1, _R), lambda i: (i, 0, 0)),
        pl.BlockSpec((1, 1, _R), lambda i: (i, 0, 0)),
        pl.BlockSpec((6, 16), lambda i: (0, 0)),
        pl.BlockSpec((5, 16), lambda i: (0, 0)),
        pl.BlockSpec((16, 1), lambda i: (0, 0)),
        pl.BlockSpec((16, 1), lambda i: (0, 0)),
        pl.BlockSpec((16, 16), lambda i: (0, 0)),
        pl.BlockSpec((16, 1), lambda i: (0, 0)),
        pl.BlockSpec((112, h_dim), lambda i: (0, 0)),
        pl.BlockSpec((h_dim, 1), lambda i: (0, 0)),
        pl.BlockSpec((h_dim, h_dim), lambda i: (0, 0)),
        pl.BlockSpec((1, h_dim), lambda i: (0, 0)),
    ]
    out_specs = pl.BlockSpec((_R, h_dim), lambda i: (i, 0))
    return in_specs, out_specs


def kernel(actor_ids, role_types, status_ids, screen_time, actor_table,
           role_table, status_table, W1, b1, W2, b2, W3, b3, W4, b4):
    b_sz, l_sz = actor_ids.shape
    n = b_sz * l_sz
    d = actor_table.shape[1]
    h_dim = W4.shape[1]
    nblk = n // _R

    ids_t = actor_ids.T.astype(jnp.int32)
    table2 = _pair_table(actor_table)
    ids_2d = (ids_t - _SPLIT * (ids_t >= _SPLIT)).reshape(n // _IDX_LANES,
                                                          _IDX_LANES)
    actor_emb = _sc_gather(ids_2d, table2, n)

    aid3 = ids_t.reshape(nblk, 1, _R)
    rid3 = role_types.T.astype(jnp.int32).reshape(nblk, 1, _R)
    sid3 = status_ids.T.astype(jnp.int32).reshape(nblk, 1, _R)
    st3 = screen_time.T.astype(jnp.float32).reshape(nblk, 1, _R)

    in_specs, out_specs = _tc_specs(nblk, h_dim)
    out = pl.pallas_call(
        _tc_body,
        grid=(nblk,),
        in_specs=in_specs,
        out_specs=out_specs,
        out_shape=jax.ShapeDtypeStruct((n, h_dim), jnp.float32),
    )(actor_emb, aid3, rid3, sid3, st3, role_table, status_table,
      W1.reshape(d // 4, 1), b1.reshape(d // 4, 1), W2,
      b2.reshape(d // 4, 1), W3, b3.reshape(h_dim, 1), W4,
      b4.reshape(1, h_dim))
    return out.reshape(l_sz, b_sz, h_dim).transpose(1, 0, 2)
